# Initial kernel scaffold; baseline (speedup 1.0000x reference)
import jax
import jax.numpy as jnp
from jax.experimental import pallas as pl


def kernel(feat, edge_index, efeat, W_edge, b_edge, W_ih, W_hh, b_ih, b_hh):
    raise NotImplementedError("write your pallas kernel here")


# speedup vs baseline: 3.8273x; 3.8273x over previous
import functools

import jax
import jax.numpy as jnp
import numpy as np
from jax import lax
from jax.experimental import pallas as pl
from jax.experimental.pallas import tpu as pltpu
from jax.experimental.pallas import tpu_sc as plsc

N = 10000
E = 320000
F = 16
NC = 2
NS = 16
NW = NC * NS
EPW = E // NW
CH = 2000
NCH = EPW // CH
NPS = N // NS

_R_np = np.zeros((F, F * F), np.float32)
for _i in range(F):
    _R_np[_i, _i * F:(_i + 1) * F] = 1.0
_S_np = np.tile(np.eye(F, dtype=np.float32), (F, 1))


def _gather_body(h_hbm, src_hbm, out_hbm, idx_v, rows_v, sem):
    c = lax.axis_index("c")
    s = lax.axis_index("s")
    base = (c * NS + s) * EPW
    for k in range(NCH):
        off = base + k * CH
        pltpu.sync_copy(src_hbm.at[pl.ds(off, CH)], idx_v)
        pltpu.async_copy(h_hbm.at[idx_v], rows_v, sem).wait()
        pltpu.sync_copy(rows_v, out_hbm.at[pl.ds(off, CH)])


@functools.lru_cache(maxsize=None)
def _sc_gather():
    return pl.kernel(
        _gather_body,
        out_type=jax.ShapeDtypeStruct((E, F), jnp.float32),
        mesh=plsc.VectorSubcoreMesh(core_axis_name="c", subcore_axis_name="s"),
        scratch_types=[
            pltpu.VMEM((CH,), jnp.int32),
            pltpu.VMEM((CH, F), jnp.float32),
            pltpu.SemaphoreType.DMA,
        ],
        compiler_params=pltpu.CompilerParams(use_tc_tiling_on_sc=False),
    )


def _scatter_body(m_hbm, dst_hbm, zeros_hbm, out_hbm, idx_v, rows_v, acc):
    c = lax.axis_index("c")
    s = lax.axis_index("s")
    pltpu.sync_copy(zeros_hbm.at[pl.ds(s * NPS, NPS)], acc.at[pl.ds(s * NPS, NPS)])
    plsc.subcore_barrier()
    base = (c * NS + s) * EPW
    for k in range(NCH):
        off = base + k * CH
        pltpu.sync_copy(dst_hbm.at[pl.ds(off, CH)], idx_v)
        pltpu.sync_copy(m_hbm.at[pl.ds(off, CH)], rows_v)
        pltpu.sync_copy(rows_v, acc.at[idx_v], add=True)
    plsc.subcore_barrier()
    pltpu.sync_copy(acc.at[pl.ds(s * NPS, NPS)], out_hbm.at[c, pl.ds(s * NPS, NPS)])


@functools.lru_cache(maxsize=None)
def _sc_scatter():
    return pl.kernel(
        _scatter_body,
        out_type=jax.ShapeDtypeStruct((NC, N, F), jnp.float32),
        mesh=plsc.VectorSubcoreMesh(core_axis_name="c", subcore_axis_name="s"),
        scratch_types=[
            pltpu.VMEM((CH,), jnp.int32),
            pltpu.VMEM((CH, F), jnp.float32),
            pltpu.VMEM_SHARED((N, F), jnp.float32),
        ],
        compiler_params=pltpu.CompilerParams(use_tc_tiling_on_sc=False),
    )


BM = 2000


def _msg_body(hsrc_ref, efeat_ref, We_ref, be_ref, R_ref, S_ref, out_ref):
    wf = jnp.dot(efeat_ref[...], We_ref[...], preferred_element_type=jnp.float32)
    wf = wf + be_ref[...]
    hexp = jnp.dot(hsrc_ref[...], R_ref[...], preferred_element_type=jnp.float32)
    out_ref[...] = jnp.dot(hexp * wf, S_ref[...], preferred_element_type=jnp.float32)


def _messages(h_src, efeat, W_edge, b_edge2, Rm, Sm):
    return pl.pallas_call(
        _msg_body,
        grid=(E // BM,),
        in_specs=[
            pl.BlockSpec((BM, F), lambda i: (i, 0)),
            pl.BlockSpec((BM, F), lambda i: (i, 0)),
            pl.BlockSpec((F, F * F), lambda i: (0, 0)),
            pl.BlockSpec((1, F * F), lambda i: (0, 0)),
            pl.BlockSpec((F, F * F), lambda i: (0, 0)),
            pl.BlockSpec((F * F, F), lambda i: (0, 0)),
        ],
        out_specs=pl.BlockSpec((BM, F), lambda i: (i, 0)),
        out_shape=jax.ShapeDtypeStruct((E, F), jnp.float32),
    )(h_src, efeat, W_edge, b_edge2, Rm, Sm)


def _gru_body(rst2_ref, h_ref, WihT_ref, WhhT_ref, bih_ref, bhh_ref, out_ref):
    rst = rst2_ref[0] + rst2_ref[1]
    h = h_ref[...]
    gi = jnp.dot(rst, WihT_ref[...], preferred_element_type=jnp.float32) + bih_ref[...]
    gh = jnp.dot(h, WhhT_ref[...], preferred_element_type=jnp.float32) + bhh_ref[...]
    r = jax.nn.sigmoid(gi[:, 0:F] + gh[:, 0:F])
    z = jax.nn.sigmoid(gi[:, F:2 * F] + gh[:, F:2 * F])
    n = jnp.tanh(gi[:, 2 * F:3 * F] + r * gh[:, 2 * F:3 * F])
    out_ref[...] = (1.0 - z) * n + z * h


def _gru(rst2, h, WihT, WhhT, bih2, bhh2):
    return pl.pallas_call(
        _gru_body,
        out_shape=jax.ShapeDtypeStruct((N, F), jnp.float32),
    )(rst2, h, WihT, WhhT, bih2, bhh2)


def kernel(feat, edge_index, efeat, W_edge, b_edge, W_ih, W_hh, b_ih, b_hh):
    src = edge_index[0]
    dst = edge_index[1]
    Rm = jnp.asarray(_R_np)
    Sm = jnp.asarray(_S_np)
    b_edge2 = b_edge.reshape(1, F * F)
    WihT = W_ih.T
    WhhT = W_hh.T
    bih2 = b_ih.reshape(1, 3 * F)
    bhh2 = b_hh.reshape(1, 3 * F)
    zeros_nf = jnp.zeros((N, F), jnp.float32)
    h = feat
    for _ in range(2):
        h_src = _sc_gather()(h, src)
        m = _messages(h_src, efeat, W_edge, b_edge2, Rm, Sm)
        rst2 = _sc_scatter()(m, dst, zeros_nf)
        h = _gru(rst2, h, WihT, WhhT, bih2, bhh2)
    return h


# speedup vs baseline: 7.9533x; 2.0780x over previous
import functools

import jax
import jax.numpy as jnp
import numpy as np
from jax import lax
from jax.experimental import pallas as pl
from jax.experimental.pallas import tpu as pltpu
from jax.experimental.pallas import tpu_sc as plsc

N = 10000
E = 320000
F = 16
NC = 2
NS = 16
NW = NC * NS
EPW = E // NW
CH = 2000
NCH = EPW // CH
NPS = N // NS
NP = N // 8
EP = E // 8

_R_np = np.zeros((F, F * F), np.float32)
for _i in range(F):
    _R_np[_i, _i * F:(_i + 1) * F] = 1.0
_S_np = np.tile(np.eye(F, dtype=np.float32), (F, 1))


def _unpack8(x):
    b = x.shape[0]
    return jnp.concatenate([x[:, 16 * a:16 * (a + 1)] for a in range(8)], axis=0)


def _pack8(m):
    b = m.shape[0] // 8
    return jnp.concatenate([m[a * b:(a + 1) * b, :] for a in range(8)], axis=1)


def _gather_body(h_hbm, src_hbm, out_hbm, idx_v, rows_v, sem):
    c = lax.axis_index("c")
    s = lax.axis_index("s")
    w = c * NS + s
    for k in range(NCH):
        off = w * EPW + k * CH
        pltpu.sync_copy(src_hbm.at[pl.ds(off, CH)], idx_v)
        pltpu.async_copy(h_hbm.at[idx_v], rows_v, sem).wait()
        pltpu.sync_copy(rows_v, out_hbm.at[w * NCH + k])


@functools.lru_cache(maxsize=None)
def _sc_gather():
    return pl.kernel(
        _gather_body,
        out_type=jax.ShapeDtypeStruct((NW * NCH, CH, F), jnp.float32),
        mesh=plsc.VectorSubcoreMesh(core_axis_name="c", subcore_axis_name="s"),
        scratch_types=[
            pltpu.VMEM((CH,), jnp.int32),
            pltpu.VMEM((CH, F), jnp.float32),
            pltpu.SemaphoreType.DMA,
        ],
        compiler_params=pltpu.CompilerParams(use_tc_tiling_on_sc=False),
    )


def _scatter_body(m_hbm, dst_hbm, zeros_hbm, out_hbm, idx_v, rows_v, acc):
    c = lax.axis_index("c")
    s = lax.axis_index("s")
    pltpu.sync_copy(zeros_hbm.at[pl.ds(s * NPS, NPS)], acc.at[pl.ds(s * NPS, NPS)])
    plsc.subcore_barrier()
    w = c * NS + s
    for k in range(NCH):
        off = w * EPW + k * CH
        pltpu.sync_copy(dst_hbm.at[pl.ds(off, CH)], idx_v)
        pltpu.sync_copy(m_hbm.at[w * NCH + k], rows_v)
        pltpu.sync_copy(rows_v, acc.at[idx_v], add=True)
    plsc.subcore_barrier()
    pltpu.sync_copy(acc.at[pl.ds(s * NPS, NPS)], out_hbm.at[c, pl.ds(s * NPS, NPS)])


@functools.lru_cache(maxsize=None)
def _sc_scatter():
    return pl.kernel(
        _scatter_body,
        out_type=jax.ShapeDtypeStruct((NC, N, F), jnp.float32),
        mesh=plsc.VectorSubcoreMesh(core_axis_name="c", subcore_axis_name="s"),
        scratch_types=[
            pltpu.VMEM((CH,), jnp.int32),
            pltpu.VMEM((CH, F), jnp.float32),
            pltpu.VMEM_SHARED((N, F), jnp.float32),
        ],
        compiler_params=pltpu.CompilerParams(use_tc_tiling_on_sc=False),
    )


BMP = 1000
BM = BMP * 8


def _msg_body(hsrcp_ref, efp_ref, We_ref, be_ref, R_ref, S_ref, mp_ref):
    y_h = _unpack8(hsrcp_ref[...])
    y_ef = _unpack8(efp_ref[...])
    wf = jnp.dot(y_ef, We_ref[...], preferred_element_type=jnp.float32)
    wf = wf + be_ref[...]
    hexp = jnp.dot(y_h, R_ref[...], preferred_element_type=jnp.float32)
    m16 = jnp.dot(hexp * wf, S_ref[...], preferred_element_type=jnp.float32)
    mp_ref[...] = _pack8(m16)


def _messages(h_srcp, efeat_p, W_edge, b_edge2, Rm, Sm):
    return pl.pallas_call(
        _msg_body,
        grid=(EP // BMP,),
        in_specs=[
            pl.BlockSpec((BMP, 128), lambda i: (i, 0)),
            pl.BlockSpec((BMP, 128), lambda i: (i, 0)),
            pl.BlockSpec((F, F * F), lambda i: (0, 0)),
            pl.BlockSpec((1, F * F), lambda i: (0, 0)),
            pl.BlockSpec((F, F * F), lambda i: (0, 0)),
            pl.BlockSpec((F * F, F), lambda i: (0, 0)),
        ],
        out_specs=pl.BlockSpec((BMP, 128), lambda i: (i, 0)),
        out_shape=jax.ShapeDtypeStruct((EP, 128), jnp.float32),
    )(h_srcp, efeat_p, W_edge, b_edge2, Rm, Sm)


def _gru_body(rst2p_ref, hp_ref, WihT_ref, WhhT_ref, bih_ref, bhh_ref, outp_ref):
    rst_p = rst2p_ref[0:NP, :] + rst2p_ref[NP:2 * NP, :]
    rst = _unpack8(rst_p)
    h = _unpack8(hp_ref[...])
    gi = jnp.dot(rst, WihT_ref[...], preferred_element_type=jnp.float32) + bih_ref[...]
    gh = jnp.dot(h, WhhT_ref[...], preferred_element_type=jnp.float32) + bhh_ref[...]
    r = jax.nn.sigmoid(gi[:, 0:F] + gh[:, 0:F])
    z = jax.nn.sigmoid(gi[:, F:2 * F] + gh[:, F:2 * F])
    n = jnp.tanh(gi[:, 2 * F:3 * F] + r * gh[:, 2 * F:3 * F])
    outp_ref[...] = _pack8((1.0 - z) * n + z * h)


def _gru(rst2_p, h_p, WihT, WhhT, bih2, bhh2):
    return pl.pallas_call(
        _gru_body,
        out_shape=jax.ShapeDtypeStruct((NP, 128), jnp.float32),
    )(rst2_p, h_p, WihT, WhhT, bih2, bhh2)


def kernel(feat, edge_index, efeat, W_edge, b_edge, W_ih, W_hh, b_ih, b_hh):
    src = edge_index[0]
    dst = edge_index[1]
    Rm = jnp.asarray(_R_np)
    Sm = jnp.asarray(_S_np)
    b_edge2 = b_edge.reshape(1, F * F)
    WihT = W_ih.T
    WhhT = W_hh.T
    bih2 = b_ih.reshape(1, 3 * F)
    bhh2 = b_hh.reshape(1, 3 * F)
    zeros_nf = jnp.zeros((N, F), jnp.float32)
    efeat_p = efeat.reshape(EP, 128)
    h_p = feat.reshape(NP, 128)
    for _ in range(2):
        h_lin = h_p.reshape(N, F)
        h_src3 = _sc_gather()(h_lin, src)
        h_srcp = h_src3.reshape(EP, 128)
        m_p = _messages(h_srcp, efeat_p, W_edge, b_edge2, Rm, Sm)
        m3 = m_p.reshape(NW * NCH, CH, F)
        rst2 = _sc_scatter()(m3, dst, zeros_nf)
        rst2_p = rst2.reshape(2 * NP, 128)
        h_p = _gru(rst2_p, h_p, WihT, WhhT, bih2, bhh2)
    return h_p.reshape(N, F)
